# baseline (speedup 1.0000x reference)
import functools

import jax
import jax.numpy as jnp
from jax import lax
from jax.experimental import pallas as pl
from jax.experimental.pallas import tpu as pltpu
from jax.experimental.pallas import tpu_sc as plsc

H, CO = 4, 32
ROW = 144
NC, NS = 1, 16
CHUNK = 128


def _prep_body(x_ref, wlt_ref, asrc_ref, adst_ref, gsum_ref, node_ref, adst_out_ref):
    xt = jnp.dot(x_ref[...], wlt_ref[...], preferred_element_type=jnp.float32)
    a16 = jnp.dot(xt * asrc_ref[...], gsum_ref[...], preferred_element_type=jnp.float32)
    node_ref[...] = jnp.concatenate([xt, a16], axis=1)
    adst_out_ref[...] = jnp.dot(xt * adst_ref[...], gsum_ref[...],
                                preferred_element_type=jnp.float32)


def _ae_body(ea_ref, me_ref, out_ref):
    out_ref[...] = jnp.dot(ea_ref[...], me_ref[...], preferred_element_type=jnp.float32)


def _sc_body(e_per_tile, node_hbm, adst_hbm, ae_hbm, src_hbm, dst_hbm, out_hbm,
             acc_sh, rows_v, adst_v, ae_v, src_v, dst_v, sem_a, sem_b):
    c = lax.axis_index("c")
    s = lax.axis_index("s")
    wid = c * NS + s
    n_pad = acc_sh.shape[0]
    rows_per_tile = n_pad // NS
    n_chunks = e_per_tile // CHUNK

    def _zero_row(i, _):
        for j in range(ROW // 16):
            rows_v[i, pl.ds(16 * j, 16)] = jnp.zeros((16,), jnp.float32)
        return 0
    lax.fori_loop(0, CHUNK, _zero_row, 0)
    for k in range(rows_per_tile // CHUNK):
        pltpu.sync_copy(rows_v, acc_sh.at[pl.ds(s * rows_per_tile + k * CHUNK, CHUNK)])
    plsc.subcore_barrier()

    def _chunk(t, _):
        base = wid * e_per_tile + t * CHUNK
        pltpu.sync_copy(src_hbm.at[pl.ds(base, CHUNK)], src_v)
        pltpu.sync_copy(dst_hbm.at[pl.ds(base, CHUNK)], dst_v)
        pltpu.sync_copy(ae_hbm.at[pl.ds(base, CHUNK)], ae_v)
        cp1 = pltpu.async_copy(node_hbm.at[src_v], rows_v, sem_a)
        cp2 = pltpu.async_copy(adst_hbm.at[dst_v], adst_v, sem_b)
        cp1.wait()
        cp2.wait()

        def _edge(e, _):
            logit = rows_v[e, pl.ds(128, 16)] + adst_v[e, :] + ae_v[e, :]
            alpha = jnp.where(logit > 0.0, logit, 0.2 * logit)
            w = jnp.exp(alpha)
            rows_v[e, pl.ds(128, 16)] = w
            for j in range(8):
                ws = w[j // 2]
                rows_v[e, pl.ds(16 * j, 16)] = rows_v[e, pl.ds(16 * j, 16)] * ws
            return 0
        lax.fori_loop(0, CHUNK, _edge, 0)
        pltpu.sync_copy(rows_v, acc_sh.at[dst_v], add=True)
        return 0
    lax.fori_loop(0, n_chunks, _chunk, 0)
    plsc.subcore_barrier()

    for k in range(rows_per_tile // CHUNK):
        r0 = s * rows_per_tile + k * CHUNK
        pltpu.sync_copy(acc_sh.at[pl.ds(r0, CHUNK)], rows_v)
        pltpu.sync_copy(rows_v, out_hbm.at[c, pl.ds(r0, CHUNK)])


def _fin_body(parts_ref, bias_ref, g4_ref, wt0, bv0, wt1, bv1, wt2, bv2,
              lng, lnb, ho_ref, hs_ref, cs_ref):
    acc = parts_ref[0] + parts_ref[1] if parts_ref.shape[0] == 2 else parts_ref[0]
    den = jnp.dot(acc[:, 128:132], g4_ref[...], preferred_element_type=jnp.float32)
    sp = acc[:, :128] / (den + 1e-16) + bias_ref[...]

    def _lstm(inp, wt, bv):
        g = jnp.dot(inp, wt[...], preferred_element_type=jnp.float32) + bv[...]
        cc = jax.nn.sigmoid(g[:, 0:128]) * jnp.tanh(g[:, 256:384])
        hh = jax.nn.sigmoid(g[:, 384:512]) * jnp.tanh(cc)
        return hh, cc

    h1, c1 = _lstm(sp, wt0, bv0)
    h2, c2 = _lstm(h1, wt1, bv1)
    h3, c3 = _lstm(h2, wt2, bv2)
    mu = jnp.mean(h3, axis=1, keepdims=True)
    var = jnp.mean((h3 - mu) ** 2, axis=1, keepdims=True)
    ho_ref[...] = (h3 - mu) * lax.rsqrt(var + 1e-5) * lng[...] + lnb[...]
    hs_ref[0] = h1
    hs_ref[1] = h2
    hs_ref[2] = h3
    cs_ref[0] = c1
    cs_ref[1] = c2
    cs_ref[2] = c3


def kernel(x, edge_attr, W_lin, att_src, att_dst, W_edge, att_edge, bias_gat,
           W_ih0, W_hh0, b_ih0, b_hh0, W_ih1, W_hh1, b_ih1, b_hh1,
           W_ih2, W_hh2, b_ih2, b_hh2, ln_g, ln_b, edge_index):
    B, N, NF = x.shape
    E = edge_index.shape[1]
    ED = edge_attr.shape[2]
    HID = W_ih0.shape[1]
    f32 = jnp.float32

    BLK = 512
    N_PAD = -(-N // 2560) * 2560
    per_tile_rows = N_PAD // NS
    assert per_tile_rows % CHUNK == 0
    E_SELF = E + B * N
    E_PAD = -(-E_SELF // (NC * NS * CHUNK)) * (NC * NS * CHUNK)
    e_per_tile = E_PAD // (NC * NS)
    DUMMY = N + 16

    wlt = W_lin.T
    asrc_row = att_src.reshape(1, H * CO)
    adst_row = att_dst.reshape(1, H * CO)
    heads = jnp.arange(H * CO, dtype=jnp.int32) // CO
    gsum = (heads[:, None] == jnp.arange(16, dtype=jnp.int32)[None, :]).astype(f32)
    me = jnp.einsum('hcd,hc->dh', W_edge.reshape(H, CO, ED), att_edge[0])
    me16 = jnp.zeros((16, 16), f32).at[:ED, :H].set(me)
    me128 = jnp.kron(jnp.eye(8, dtype=f32), me16)
    g4 = (jnp.arange(4, dtype=jnp.int32)[:, None] == heads[None, :]).astype(f32)

    xp = jnp.zeros((N_PAD, NF), f32).at[:N].set(x.reshape(N, NF))
    loop = jnp.arange(N, dtype=edge_index.dtype)
    fill = jnp.full((E_PAD - E_SELF,), DUMMY, edge_index.dtype)
    src = jnp.concatenate([edge_index[0], loop, fill])
    dst = jnp.concatenate([edge_index[1], loop, fill])
    eap = jnp.zeros((E_PAD, 16), f32).at[:E, :ED].set(edge_attr.reshape(E, ED))

    node_tab, adst_tab = pl.pallas_call(
        _prep_body,
        grid=(N_PAD // BLK,),
        in_specs=[
            pl.BlockSpec((BLK, NF), lambda i: (i, 0)),
            pl.BlockSpec((NF, H * CO), lambda i: (0, 0)),
            pl.BlockSpec((1, H * CO), lambda i: (0, 0)),
            pl.BlockSpec((1, H * CO), lambda i: (0, 0)),
            pl.BlockSpec((H * CO, 16), lambda i: (0, 0)),
        ],
        out_specs=[
            pl.BlockSpec((BLK, ROW), lambda i: (i, 0)),
            pl.BlockSpec((BLK, 16), lambda i: (i, 0)),
        ],
        out_shape=[
            jax.ShapeDtypeStruct((N_PAD, ROW), f32),
            jax.ShapeDtypeStruct((N_PAD, 16), f32),
        ],
    )(xp, wlt, asrc_row, adst_row, gsum)

    ea_wide = eap.reshape(E_PAD // 8, 128)
    ae_tab = pl.pallas_call(
        _ae_body,
        grid=(E_PAD // 8 // BLK,),
        in_specs=[
            pl.BlockSpec((BLK, 128), lambda i: (i, 0)),
            pl.BlockSpec((128, 128), lambda i: (0, 0)),
        ],
        out_specs=pl.BlockSpec((BLK, 128), lambda i: (i, 0)),
        out_shape=jax.ShapeDtypeStruct((E_PAD // 8, 128), f32),
    )(ea_wide, me128).reshape(E_PAD, 16)

    mesh = plsc.VectorSubcoreMesh(core_axis_name="c", subcore_axis_name="s",
                                  num_cores=NC)
    sc_call = pl.kernel(
        functools.partial(_sc_body, e_per_tile),
        out_type=jax.ShapeDtypeStruct((NC, N_PAD, ROW), f32),
        mesh=mesh,
        scratch_types=[
            pltpu.VMEM_SHARED((N_PAD, ROW), f32),
            pltpu.VMEM((CHUNK, ROW), f32),
            pltpu.VMEM((CHUNK, 16), f32),
            pltpu.VMEM((CHUNK, 16), f32),
            pltpu.VMEM((CHUNK,), jnp.int32),
            pltpu.VMEM((CHUNK,), jnp.int32),
            pltpu.SemaphoreType.DMA,
            pltpu.SemaphoreType.DMA,
        ],
        compiler_params=pltpu.CompilerParams(use_tc_tiling_on_sc=False),
    )
    parts = sc_call(node_tab, adst_tab, ae_tab, src, dst)

    wt0, wt1, wt2 = W_ih0.T, W_ih1.T, W_ih2.T
    bv0 = (b_ih0 + b_hh0).reshape(1, 4 * HID)
    bv1 = (b_ih1 + b_hh1).reshape(1, 4 * HID)
    bv2 = (b_ih2 + b_hh2).reshape(1, 4 * HID)
    n_blocks = -(-N // BLK)
    ho, hs, cs = pl.pallas_call(
        _fin_body,
        grid=(n_blocks,),
        in_specs=[
            pl.BlockSpec((NC, BLK, ROW), lambda i: (0, i, 0)),
            pl.BlockSpec((1, H * CO), lambda i: (0, 0)),
            pl.BlockSpec((4, 128), lambda i: (0, 0)),
            pl.BlockSpec((HID, 4 * HID), lambda i: (0, 0)),
            pl.BlockSpec((1, 4 * HID), lambda i: (0, 0)),
            pl.BlockSpec((HID, 4 * HID), lambda i: (0, 0)),
            pl.BlockSpec((1, 4 * HID), lambda i: (0, 0)),
            pl.BlockSpec((HID, 4 * HID), lambda i: (0, 0)),
            pl.BlockSpec((1, 4 * HID), lambda i: (0, 0)),
            pl.BlockSpec((1, HID), lambda i: (0, 0)),
            pl.BlockSpec((1, HID), lambda i: (0, 0)),
        ],
        out_specs=[
            pl.BlockSpec((BLK, HID), lambda i: (i, 0)),
            pl.BlockSpec((3, BLK, HID), lambda i: (0, i, 0)),
            pl.BlockSpec((3, BLK, HID), lambda i: (0, i, 0)),
        ],
        out_shape=[
            jax.ShapeDtypeStruct((N, HID), f32),
            jax.ShapeDtypeStruct((3, N, HID), f32),
            jax.ShapeDtypeStruct((3, N, HID), f32),
        ],
    )(parts, bias_gat.reshape(1, H * CO), g4, wt0, bv0, wt1, bv1, wt2, bv2,
      ln_g.reshape(1, HID), ln_b.reshape(1, HID))

    return (ho.reshape(B, N, HID), hs, cs)


# speedup vs baseline: 1.7273x; 1.7273x over previous
import functools

import jax
import jax.numpy as jnp
from jax import lax
from jax.experimental import pallas as pl
from jax.experimental.pallas import tpu as pltpu
from jax.experimental.pallas import tpu_sc as plsc

H, CO = 4, 32
ROW = 144
NC, NS = 2, 16
CHUNK = 128


def _prep_body(x_ref, wlt_ref, asrc_ref, adst_ref, gsum_ref, node_ref, adst_out_ref):
    xt = jnp.dot(x_ref[...], wlt_ref[...], preferred_element_type=jnp.float32)
    a16 = jnp.dot(xt * asrc_ref[...], gsum_ref[...], preferred_element_type=jnp.float32)
    node_ref[...] = jnp.concatenate([xt, a16], axis=1)
    adst_out_ref[...] = jnp.dot(xt * adst_ref[...], gsum_ref[...],
                                preferred_element_type=jnp.float32)


def _ae_body(ea_ref, me_ref, out_ref):
    out_ref[...] = jnp.dot(ea_ref[...], me_ref[...], preferred_element_type=jnp.float32)


def _sc_body(e_per_tile, node_hbm, adst_hbm, ae_hbm, src_hbm, dst_hbm, out_hbm,
             acc_sh, rows_v, adst_v, ae_v, src_v, dst_v, sem_a, sem_b):
    c = lax.axis_index("c")
    s = lax.axis_index("s")
    wid = c * NS + s
    n_pad = acc_sh.shape[0]
    rows_per_tile = n_pad // NS
    n_chunks = e_per_tile // CHUNK

    @plsc.parallel_loop(0, CHUNK, unroll=4)
    def _zero_row(i):
        for j in range(ROW // 16):
            rows_v[i, pl.ds(16 * j, 16)] = jnp.zeros((16,), jnp.float32)
    for k in range(rows_per_tile // CHUNK):
        pltpu.sync_copy(rows_v, acc_sh.at[pl.ds(s * rows_per_tile + k * CHUNK, CHUNK)])
    plsc.subcore_barrier()

    def _chunk(t, _):
        base = wid * e_per_tile + t * CHUNK
        pltpu.sync_copy(src_hbm.at[pl.ds(base, CHUNK)], src_v)
        pltpu.sync_copy(dst_hbm.at[pl.ds(base, CHUNK)], dst_v)
        pltpu.sync_copy(ae_hbm.at[pl.ds(base, CHUNK)], ae_v)
        cp1 = pltpu.async_copy(node_hbm.at[src_v], rows_v, sem_a)
        cp2 = pltpu.async_copy(adst_hbm.at[dst_v], adst_v, sem_b)
        cp1.wait()
        cp2.wait()

        @plsc.parallel_loop(0, CHUNK, unroll=4)
        def _edge(e):
            logit = rows_v[e, pl.ds(128, 16)] + adst_v[e, :] + ae_v[e, :]
            w = jnp.exp(jnp.maximum(logit, 0.2 * logit))
            rows_v[e, pl.ds(128, 16)] = w
            for j in range(8):
                ws = w[j // 2]
                rows_v[e, pl.ds(16 * j, 16)] = rows_v[e, pl.ds(16 * j, 16)] * ws
        pltpu.sync_copy(rows_v, acc_sh.at[dst_v], add=True)
        return 0
    lax.fori_loop(0, n_chunks, _chunk, 0)
    plsc.subcore_barrier()

    for k in range(rows_per_tile // CHUNK):
        r0 = s * rows_per_tile + k * CHUNK
        pltpu.sync_copy(acc_sh.at[pl.ds(r0, CHUNK)], rows_v)
        pltpu.sync_copy(rows_v, out_hbm.at[c, pl.ds(r0, CHUNK)])


def _fin_body(parts_ref, bias_ref, g4_ref, wt0, bv0, wt1, bv1, wt2, bv2,
              lng, lnb, ho_ref, hs_ref, cs_ref):
    acc = parts_ref[0] + parts_ref[1] if parts_ref.shape[0] == 2 else parts_ref[0]
    den = jnp.dot(acc[:, 128:132], g4_ref[...], preferred_element_type=jnp.float32)
    sp = acc[:, :128] / (den + 1e-16) + bias_ref[...]

    def _lstm(inp, wt, bv):
        g = jnp.dot(inp, wt[...], preferred_element_type=jnp.float32) + bv[...]
        cc = jax.nn.sigmoid(g[:, 0:128]) * jnp.tanh(g[:, 256:384])
        hh = jax.nn.sigmoid(g[:, 384:512]) * jnp.tanh(cc)
        return hh, cc

    h1, c1 = _lstm(sp, wt0, bv0)
    h2, c2 = _lstm(h1, wt1, bv1)
    h3, c3 = _lstm(h2, wt2, bv2)
    mu = jnp.mean(h3, axis=1, keepdims=True)
    var = jnp.mean((h3 - mu) ** 2, axis=1, keepdims=True)
    ho_ref[...] = (h3 - mu) * lax.rsqrt(var + 1e-5) * lng[...] + lnb[...]
    hs_ref[0] = h1
    hs_ref[1] = h2
    hs_ref[2] = h3
    cs_ref[0] = c1
    cs_ref[1] = c2
    cs_ref[2] = c3


def kernel(x, edge_attr, W_lin, att_src, att_dst, W_edge, att_edge, bias_gat,
           W_ih0, W_hh0, b_ih0, b_hh0, W_ih1, W_hh1, b_ih1, b_hh1,
           W_ih2, W_hh2, b_ih2, b_hh2, ln_g, ln_b, edge_index):
    B, N, NF = x.shape
    E = edge_index.shape[1]
    ED = edge_attr.shape[2]
    HID = W_ih0.shape[1]
    f32 = jnp.float32

    BLK = 512
    N_PAD = -(-N // 2560) * 2560
    per_tile_rows = N_PAD // NS
    assert per_tile_rows % CHUNK == 0
    E_SELF = E + B * N
    E_PAD = -(-E_SELF // (NC * NS * CHUNK)) * (NC * NS * CHUNK)
    e_per_tile = E_PAD // (NC * NS)
    DUMMY = N + 16

    wlt = W_lin.T
    asrc_row = att_src.reshape(1, H * CO)
    adst_row = att_dst.reshape(1, H * CO)
    heads = jnp.arange(H * CO, dtype=jnp.int32) // CO
    gsum = (heads[:, None] == jnp.arange(16, dtype=jnp.int32)[None, :]).astype(f32)
    me = jnp.einsum('hcd,hc->dh', W_edge.reshape(H, CO, ED), att_edge[0])
    me16 = jnp.zeros((16, 16), f32).at[:ED, :H].set(me)
    me128 = jnp.kron(jnp.eye(8, dtype=f32), me16)
    g4 = (jnp.arange(4, dtype=jnp.int32)[:, None] == heads[None, :]).astype(f32)

    xp = jnp.zeros((N_PAD, NF), f32).at[:N].set(x.reshape(N, NF))
    loop = jnp.arange(N, dtype=edge_index.dtype)
    fill = jnp.full((E_PAD - E_SELF,), DUMMY, edge_index.dtype)
    src = jnp.concatenate([edge_index[0], loop, fill])
    dst = jnp.concatenate([edge_index[1], loop, fill])
    eap = jnp.zeros((E_PAD, 16), f32).at[:E, :ED].set(edge_attr.reshape(E, ED))

    node_tab, adst_tab = pl.pallas_call(
        _prep_body,
        grid=(N_PAD // BLK,),
        in_specs=[
            pl.BlockSpec((BLK, NF), lambda i: (i, 0)),
            pl.BlockSpec((NF, H * CO), lambda i: (0, 0)),
            pl.BlockSpec((1, H * CO), lambda i: (0, 0)),
            pl.BlockSpec((1, H * CO), lambda i: (0, 0)),
            pl.BlockSpec((H * CO, 16), lambda i: (0, 0)),
        ],
        out_specs=[
            pl.BlockSpec((BLK, ROW), lambda i: (i, 0)),
            pl.BlockSpec((BLK, 16), lambda i: (i, 0)),
        ],
        out_shape=[
            jax.ShapeDtypeStruct((N_PAD, ROW), f32),
            jax.ShapeDtypeStruct((N_PAD, 16), f32),
        ],
    )(xp, wlt, asrc_row, adst_row, gsum)

    ea_wide = eap.reshape(E_PAD // 8, 128)
    ae_tab = pl.pallas_call(
        _ae_body,
        grid=(E_PAD // 8 // BLK,),
        in_specs=[
            pl.BlockSpec((BLK, 128), lambda i: (i, 0)),
            pl.BlockSpec((128, 128), lambda i: (0, 0)),
        ],
        out_specs=pl.BlockSpec((BLK, 128), lambda i: (i, 0)),
        out_shape=jax.ShapeDtypeStruct((E_PAD // 8, 128), f32),
    )(ea_wide, me128).reshape(E_PAD, 16)

    mesh = plsc.VectorSubcoreMesh(core_axis_name="c", subcore_axis_name="s",
                                  num_cores=NC)
    sc_call = pl.kernel(
        functools.partial(_sc_body, e_per_tile),
        out_type=jax.ShapeDtypeStruct((NC, N_PAD, ROW), f32),
        mesh=mesh,
        scratch_types=[
            pltpu.VMEM_SHARED((N_PAD, ROW), f32),
            pltpu.VMEM((CHUNK, ROW), f32),
            pltpu.VMEM((CHUNK, 16), f32),
            pltpu.VMEM((CHUNK, 16), f32),
            pltpu.VMEM((CHUNK,), jnp.int32),
            pltpu.VMEM((CHUNK,), jnp.int32),
            pltpu.SemaphoreType.DMA,
            pltpu.SemaphoreType.DMA,
        ],
        compiler_params=pltpu.CompilerParams(use_tc_tiling_on_sc=False),
    )
    parts = sc_call(node_tab, adst_tab, ae_tab, src, dst)

    wt0, wt1, wt2 = W_ih0.T, W_ih1.T, W_ih2.T
    bv0 = (b_ih0 + b_hh0).reshape(1, 4 * HID)
    bv1 = (b_ih1 + b_hh1).reshape(1, 4 * HID)
    bv2 = (b_ih2 + b_hh2).reshape(1, 4 * HID)
    n_blocks = -(-N // BLK)
    ho, hs, cs = pl.pallas_call(
        _fin_body,
        grid=(n_blocks,),
        in_specs=[
            pl.BlockSpec((NC, BLK, ROW), lambda i: (0, i, 0)),
            pl.BlockSpec((1, H * CO), lambda i: (0, 0)),
            pl.BlockSpec((4, 128), lambda i: (0, 0)),
            pl.BlockSpec((HID, 4 * HID), lambda i: (0, 0)),
            pl.BlockSpec((1, 4 * HID), lambda i: (0, 0)),
            pl.BlockSpec((HID, 4 * HID), lambda i: (0, 0)),
            pl.BlockSpec((1, 4 * HID), lambda i: (0, 0)),
            pl.BlockSpec((HID, 4 * HID), lambda i: (0, 0)),
            pl.BlockSpec((1, 4 * HID), lambda i: (0, 0)),
            pl.BlockSpec((1, HID), lambda i: (0, 0)),
            pl.BlockSpec((1, HID), lambda i: (0, 0)),
        ],
        out_specs=[
            pl.BlockSpec((BLK, HID), lambda i: (i, 0)),
            pl.BlockSpec((3, BLK, HID), lambda i: (0, i, 0)),
            pl.BlockSpec((3, BLK, HID), lambda i: (0, i, 0)),
        ],
        out_shape=[
            jax.ShapeDtypeStruct((N, HID), f32),
            jax.ShapeDtypeStruct((3, N, HID), f32),
            jax.ShapeDtypeStruct((3, N, HID), f32),
        ],
    )(parts, bias_gat.reshape(1, H * CO), g4, wt0, bv0, wt1, bv1, wt2, bv2,
      ln_g.reshape(1, HID), ln_b.reshape(1, HID))

    return (ho.reshape(B, N, HID), hs, cs)


# speedup vs baseline: 2.3228x; 1.3448x over previous
import functools

import jax
import jax.numpy as jnp
from jax import lax
from jax.experimental import pallas as pl
from jax.experimental.pallas import tpu as pltpu
from jax.experimental.pallas import tpu_sc as plsc

H, CO = 4, 32
ROW = 144
NC, NS = 2, 16
CHUNK = 96


def _prep_body(x_ref, wlt_ref, asrc_ref, adst_ref, gsum_ref, node_ref, adst_out_ref):
    xt = jnp.dot(x_ref[...], wlt_ref[...], preferred_element_type=jnp.float32)
    a16 = jnp.dot(xt * asrc_ref[...], gsum_ref[...], preferred_element_type=jnp.float32)
    node_ref[...] = jnp.concatenate([xt, a16], axis=1)
    adst_out_ref[...] = jnp.dot(xt * adst_ref[...], gsum_ref[...],
                                preferred_element_type=jnp.float32)


def _ae_body(ea_ref, me_ref, out_ref):
    out_ref[...] = jnp.dot(ea_ref[...], me_ref[...], preferred_element_type=jnp.float32)


def _sc_body(e_per_tile, node_hbm, adst_hbm, ae_hbm, src_hbm, dst_hbm, out_hbm,
             acc_sh,
             rows0, rows1, adv0, adv1, aev0, aev1,
             siv0, siv1, div0, div1, dtv0, dtv1,
             sn0, sn1, sa0, sa1, se0, se1, sis0, sis1, sid0, sid1, ss0, ss1):
    c = lax.axis_index("c")
    s = lax.axis_index("s")
    wid = c * NS + s
    n_pad = acc_sh.shape[0]
    rows_per_tile = n_pad // NS
    n_chunks = e_per_tile // CHUNK

    rows = (rows0, rows1)
    adv = (adv0, adv1)
    aev = (aev0, aev1)
    siv = (siv0, siv1)
    div = (div0, div1)
    dtv = (dtv0, dtv1)
    sem_n = (sn0, sn1)
    sem_a = (sa0, sa1)
    sem_e = (se0, se1)
    sem_is = (sis0, sis1)
    sem_id = (sid0, sid1)
    sem_s = (ss0, ss1)

    @plsc.parallel_loop(0, CHUNK, unroll=4)
    def _zero_row(i):
        for j in range(ROW // 16):
            rows0[i, pl.ds(16 * j, 16)] = jnp.zeros((16,), jnp.float32)
    r0 = s * rows_per_tile
    nfull, rem = divmod(rows_per_tile, CHUNK)
    for k in range(nfull):
        pltpu.sync_copy(rows0, acc_sh.at[pl.ds(r0 + k * CHUNK, CHUNK)])
    if rem:
        pltpu.sync_copy(rows0.at[pl.ds(0, rem)],
                        acc_sh.at[pl.ds(r0 + nfull * CHUNK, rem)])
    plsc.subcore_barrier()

    def _base(t):
        return wid * e_per_tile + t * CHUNK

    def _start_idx(t, b):
        pltpu.async_copy(src_hbm.at[pl.ds(_base(t), CHUNK)], siv[b], sem_is[b])
        pltpu.async_copy(dst_hbm.at[pl.ds(_base(t), CHUNK)], div[b], sem_id[b])

    def _wait_idx(t, b):
        pltpu.make_async_copy(src_hbm.at[pl.ds(_base(t), CHUNK)], siv[b],
                              sem_is[b]).wait()
        pltpu.make_async_copy(dst_hbm.at[pl.ds(_base(t), CHUNK)], div[b],
                              sem_id[b]).wait()

    def _start_gathers(t, b):
        pltpu.async_copy(node_hbm.at[siv[b]], rows[b], sem_n[b])
        pltpu.async_copy(adst_hbm.at[div[b]], adv[b], sem_a[b])
        pltpu.async_copy(ae_hbm.at[pl.ds(_base(t), CHUNK)], aev[b], sem_e[b])

    def _wait_gathers(t, b):
        pltpu.make_async_copy(node_hbm.at[siv[b]], rows[b], sem_n[b]).wait()
        pltpu.make_async_copy(adst_hbm.at[div[b]], adv[b], sem_a[b]).wait()
        pltpu.make_async_copy(ae_hbm.at[pl.ds(_base(t), CHUNK)], aev[b],
                              sem_e[b]).wait()

    def _wait_scat(b):
        pltpu.make_async_copy(rows[b], acc_sh.at[dtv[b]], sem_s[b]).wait()

    def _chunk_body(t, b, start_idx2, start_next, wait_prev_scat):
        rows_v, adst_v, ae_v = rows[b], adv[b], aev[b]
        _wait_gathers(t, b)
        for k in range(CHUNK // 16):
            dtv[b][pl.ds(16 * k, 16)] = div[b][pl.ds(16 * k, 16)]
        if start_idx2:
            _start_idx(t + 2, b)

        @plsc.parallel_loop(0, CHUNK, unroll=4)
        def _edge(e):
            logit = rows_v[e, pl.ds(128, 16)] + adst_v[e, :] + ae_v[e, :]
            w = jnp.exp(jnp.maximum(logit, 0.2 * logit))
            rows_v[e, pl.ds(128, 16)] = w
            for j in range(8):
                ws = w[j // 2]
                rows_v[e, pl.ds(16 * j, 16)] = rows_v[e, pl.ds(16 * j, 16)] * ws

        pltpu.async_copy(rows_v, acc_sh.at[dtv[b]], sem_s[b], add=True)
        if start_next:
            if wait_prev_scat:
                _wait_scat(1 - b)
            _wait_idx(t + 1, 1 - b)
            _start_gathers(t + 1, 1 - b)

    def _pair(p, _):
        t = 2 * p + 2
        _chunk_body(t, 0, True, True, True)
        _chunk_body(t + 1, 1, True, True, True)
        return 0

    _start_idx(0, 0)
    _start_idx(1, 1)
    _wait_idx(0, 0)
    _start_gathers(0, 0)
    _chunk_body(0, 0, True, True, False)
    _chunk_body(1, 1, True, True, True)
    lax.fori_loop(0, (n_chunks - 4) // 2, _pair, 0)
    _chunk_body(n_chunks - 2, 0, False, True, True)
    _chunk_body(n_chunks - 1, 1, False, False, False)
    for b in range(2):
        _wait_scat(b)
    plsc.subcore_barrier()

    for k in range(nfull):
        rr = r0 + k * CHUNK
        pltpu.sync_copy(acc_sh.at[pl.ds(rr, CHUNK)], rows0)
        pltpu.sync_copy(rows0, out_hbm.at[c, pl.ds(rr, CHUNK)])
    if rem:
        rr = r0 + nfull * CHUNK
        pltpu.sync_copy(acc_sh.at[pl.ds(rr, rem)], rows0.at[pl.ds(0, rem)])
        pltpu.sync_copy(rows0.at[pl.ds(0, rem)], out_hbm.at[c, pl.ds(rr, rem)])


def _fin_body(parts_ref, bias_ref, g4_ref, wt0, bv0, wt1, bv1, wt2, bv2,
              lng, lnb, ho_ref, hs_ref, cs_ref):
    acc = parts_ref[0] + parts_ref[1] if parts_ref.shape[0] == 2 else parts_ref[0]
    den = jnp.dot(acc[:, 128:132], g4_ref[...], preferred_element_type=jnp.float32)
    sp = acc[:, :128] / (den + 1e-16) + bias_ref[...]

    def _lstm(inp, wt, bv):
        g = jnp.dot(inp, wt[...], preferred_element_type=jnp.float32) + bv[...]
        cc = jax.nn.sigmoid(g[:, 0:128]) * jnp.tanh(g[:, 256:384])
        hh = jax.nn.sigmoid(g[:, 384:512]) * jnp.tanh(cc)
        return hh, cc

    h1, c1 = _lstm(sp, wt0, bv0)
    h2, c2 = _lstm(h1, wt1, bv1)
    h3, c3 = _lstm(h2, wt2, bv2)
    mu = jnp.mean(h3, axis=1, keepdims=True)
    var = jnp.mean((h3 - mu) ** 2, axis=1, keepdims=True)
    ho_ref[...] = (h3 - mu) * lax.rsqrt(var + 1e-5) * lng[...] + lnb[...]
    hs_ref[0] = h1
    hs_ref[1] = h2
    hs_ref[2] = h3
    cs_ref[0] = c1
    cs_ref[1] = c2
    cs_ref[2] = c3


def kernel(x, edge_attr, W_lin, att_src, att_dst, W_edge, att_edge, bias_gat,
           W_ih0, W_hh0, b_ih0, b_hh0, W_ih1, W_hh1, b_ih1, b_hh1,
           W_ih2, W_hh2, b_ih2, b_hh2, ln_g, ln_b, edge_index):
    B, N, NF = x.shape
    E = edge_index.shape[1]
    ED = edge_attr.shape[2]
    HID = W_ih0.shape[1]
    f32 = jnp.float32

    BLK = 512
    N_PAD = -(-N // 2560) * 2560
    E_SELF = E + B * N
    E_PAD = -(-E_SELF // (NC * NS * CHUNK * 2)) * (NC * NS * CHUNK * 2)
    e_per_tile = E_PAD // (NC * NS)
    n_chunks = e_per_tile // CHUNK

    wlt = W_lin.T
    asrc_row = att_src.reshape(1, H * CO)
    adst_row = att_dst.reshape(1, H * CO)
    heads = jnp.arange(H * CO, dtype=jnp.int32) // CO
    gsum = (heads[:, None] == jnp.arange(16, dtype=jnp.int32)[None, :]).astype(f32)
    me = jnp.einsum('hcd,hc->dh', W_edge.reshape(H, CO, ED), att_edge[0])
    me16 = jnp.zeros((16, 16), f32).at[:ED, :H].set(me)
    me128 = jnp.kron(jnp.eye(8, dtype=f32), me16)
    g4 = (jnp.arange(4, dtype=jnp.int32)[:, None] == heads[None, :]).astype(f32)

    xp = jnp.zeros((N_PAD, NF), f32).at[:N].set(x.reshape(N, NF))
    loop = jnp.arange(N, dtype=edge_index.dtype)
    n_fill = E_PAD - E_SELF
    fill = (N + jnp.arange(n_fill, dtype=edge_index.dtype) % (N_PAD - N - 8))
    src = jnp.concatenate([edge_index[0], loop, fill])
    dst = jnp.concatenate([edge_index[1], loop, fill])
    eap = jnp.zeros((E_PAD, 16), f32).at[:E, :ED].set(edge_attr.reshape(E, ED))

    node_tab, adst_tab = pl.pallas_call(
        _prep_body,
        grid=(N_PAD // BLK,),
        in_specs=[
            pl.BlockSpec((BLK, NF), lambda i: (i, 0)),
            pl.BlockSpec((NF, H * CO), lambda i: (0, 0)),
            pl.BlockSpec((1, H * CO), lambda i: (0, 0)),
            pl.BlockSpec((1, H * CO), lambda i: (0, 0)),
            pl.BlockSpec((H * CO, 16), lambda i: (0, 0)),
        ],
        out_specs=[
            pl.BlockSpec((BLK, ROW), lambda i: (i, 0)),
            pl.BlockSpec((BLK, 16), lambda i: (i, 0)),
        ],
        out_shape=[
            jax.ShapeDtypeStruct((N_PAD, ROW), f32),
            jax.ShapeDtypeStruct((N_PAD, 16), f32),
        ],
    )(xp, wlt, asrc_row, adst_row, gsum)

    ea_wide = eap.reshape(E_PAD // 8, 128)
    ae_tab = pl.pallas_call(
        _ae_body,
        grid=(E_PAD // 8 // BLK,),
        in_specs=[
            pl.BlockSpec((BLK, 128), lambda i: (i, 0)),
            pl.BlockSpec((128, 128), lambda i: (0, 0)),
        ],
        out_specs=pl.BlockSpec((BLK, 128), lambda i: (i, 0)),
        out_shape=jax.ShapeDtypeStruct((E_PAD // 8, 128), f32),
    )(ea_wide, me128).reshape(E_PAD, 16)

    mesh = plsc.VectorSubcoreMesh(core_axis_name="c", subcore_axis_name="s",
                                  num_cores=NC)
    sc_call = pl.kernel(
        functools.partial(_sc_body, e_per_tile),
        out_type=jax.ShapeDtypeStruct((NC, N_PAD, ROW), f32),
        mesh=mesh,
        scratch_types=(
            [pltpu.VMEM_SHARED((N_PAD, ROW), f32)]
            + [pltpu.VMEM((CHUNK, ROW), f32)] * 2
            + [pltpu.VMEM((CHUNK, 16), f32)] * 4
            + [pltpu.VMEM((CHUNK,), jnp.int32)] * 6
            + [pltpu.SemaphoreType.DMA] * 12
        ),
        compiler_params=pltpu.CompilerParams(use_tc_tiling_on_sc=False),
    )
    parts = sc_call(node_tab, adst_tab, ae_tab, src, dst)

    wt0, wt1, wt2 = W_ih0.T, W_ih1.T, W_ih2.T
    bv0 = (b_ih0 + b_hh0).reshape(1, 4 * HID)
    bv1 = (b_ih1 + b_hh1).reshape(1, 4 * HID)
    bv2 = (b_ih2 + b_hh2).reshape(1, 4 * HID)
    n_blocks = -(-N // BLK)
    ho, hs, cs = pl.pallas_call(
        _fin_body,
        grid=(n_blocks,),
        in_specs=[
            pl.BlockSpec((NC, BLK, ROW), lambda i: (0, i, 0)),
            pl.BlockSpec((1, H * CO), lambda i: (0, 0)),
            pl.BlockSpec((4, 128), lambda i: (0, 0)),
            pl.BlockSpec((HID, 4 * HID), lambda i: (0, 0)),
            pl.BlockSpec((1, 4 * HID), lambda i: (0, 0)),
            pl.BlockSpec((HID, 4 * HID), lambda i: (0, 0)),
            pl.BlockSpec((1, 4 * HID), lambda i: (0, 0)),
            pl.BlockSpec((HID, 4 * HID), lambda i: (0, 0)),
            pl.BlockSpec((1, 4 * HID), lambda i: (0, 0)),
            pl.BlockSpec((1, HID), lambda i: (0, 0)),
            pl.BlockSpec((1, HID), lambda i: (0, 0)),
        ],
        out_specs=[
            pl.BlockSpec((BLK, HID), lambda i: (i, 0)),
            pl.BlockSpec((3, BLK, HID), lambda i: (0, i, 0)),
            pl.BlockSpec((3, BLK, HID), lambda i: (0, i, 0)),
        ],
        out_shape=[
            jax.ShapeDtypeStruct((N, HID), f32),
            jax.ShapeDtypeStruct((3, N, HID), f32),
            jax.ShapeDtypeStruct((3, N, HID), f32),
        ],
    )(parts, bias_gat.reshape(1, H * CO), g4, wt0, bv0, wt1, bv1, wt2, bv2,
      ln_g.reshape(1, HID), ln_b.reshape(1, HID))

    return (ho.reshape(B, N, HID), hs, cs)


# speedup vs baseline: 2.3993x; 1.0329x over previous
import functools

import jax
import jax.numpy as jnp
from jax import lax
from jax.experimental import pallas as pl
from jax.experimental.pallas import tpu as pltpu
from jax.experimental.pallas import tpu_sc as plsc

H, CO = 4, 32
ROW = 144
NC, NS = 2, 16
CHUNK = 96


def _prep_body(x_ref, wlt_ref, asrc_ref, adst_ref, gsum_ref, node_ref, adst_out_ref):
    xt = jnp.dot(x_ref[...], wlt_ref[...], preferred_element_type=jnp.float32)
    a16 = jnp.dot(xt * asrc_ref[...], gsum_ref[...], preferred_element_type=jnp.float32)
    node_ref[...] = jnp.concatenate([xt, a16], axis=1)
    adst_out_ref[...] = jnp.dot(xt * adst_ref[...], gsum_ref[...],
                                preferred_element_type=jnp.float32)


def _ae_body(ea_ref, me_ref, out_ref):
    out_ref[...] = jnp.dot(ea_ref[...], me_ref[...], preferred_element_type=jnp.float32)


def _sc_body(e_per_tile, node_hbm, adst_hbm, ae_hbm, src_hbm, dst_hbm, out_hbm,
             acc_sh,
             rows0, rows1, adv0, adv1, aev0, aev1,
             siv0, siv1, div0, div1, dtv0, dtv1,
             sn0, sn1, sa0, sa1, se0, se1, sis0, sis1, sid0, sid1, ss0, ss1):
    c = lax.axis_index("c")
    s = lax.axis_index("s")
    wid = c * NS + s
    n_pad = acc_sh.shape[0]
    rows_per_tile = n_pad // NS
    n_chunks = e_per_tile // CHUNK

    rows = (rows0, rows1)
    adv = (adv0, adv1)
    aev = (aev0, aev1)
    siv = (siv0, siv1)
    div = (div0, div1)
    dtv = (dtv0, dtv1)
    sem_n = (sn0, sn1)
    sem_a = (sa0, sa1)
    sem_e = (se0, se1)
    sem_is = (sis0, sis1)
    sem_id = (sid0, sid1)
    sem_s = (ss0, ss1)

    @plsc.parallel_loop(0, CHUNK, unroll=4)
    def _zero_row(i):
        for j in range(ROW // 16):
            rows0[i, pl.ds(16 * j, 16)] = jnp.zeros((16,), jnp.float32)
    r0 = s * rows_per_tile
    nfull, rem = divmod(rows_per_tile, CHUNK)
    for k in range(nfull):
        pltpu.sync_copy(rows0, acc_sh.at[pl.ds(r0 + k * CHUNK, CHUNK)])
    if rem:
        pltpu.sync_copy(rows0.at[pl.ds(0, rem)],
                        acc_sh.at[pl.ds(r0 + nfull * CHUNK, rem)])
    plsc.subcore_barrier()

    def _base(t):
        return wid * e_per_tile + t * CHUNK

    def _start_idx(t, b):
        pltpu.async_copy(src_hbm.at[pl.ds(_base(t), CHUNK)], siv[b], sem_is[b])
        pltpu.async_copy(dst_hbm.at[pl.ds(_base(t), CHUNK)], div[b], sem_id[b])

    def _wait_idx(t, b):
        pltpu.make_async_copy(src_hbm.at[pl.ds(_base(t), CHUNK)], siv[b],
                              sem_is[b]).wait()
        pltpu.make_async_copy(dst_hbm.at[pl.ds(_base(t), CHUNK)], div[b],
                              sem_id[b]).wait()

    def _start_gathers(t, b):
        pltpu.async_copy(node_hbm.at[siv[b]], rows[b], sem_n[b])
        pltpu.async_copy(adst_hbm.at[div[b]], adv[b], sem_a[b])
        pltpu.async_copy(ae_hbm.at[pl.ds(_base(t), CHUNK)], aev[b], sem_e[b])

    def _wait_gathers(t, b):
        pltpu.make_async_copy(node_hbm.at[siv[b]], rows[b], sem_n[b]).wait()
        pltpu.make_async_copy(adst_hbm.at[div[b]], adv[b], sem_a[b]).wait()
        pltpu.make_async_copy(ae_hbm.at[pl.ds(_base(t), CHUNK)], aev[b],
                              sem_e[b]).wait()

    def _wait_scat(b):
        pltpu.make_async_copy(rows[b], acc_sh.at[dtv[b]], sem_s[b]).wait()

    def _chunk_body(t, b, start_idx2, start_next, wait_prev_scat):
        rows_v, adst_v, ae_v = rows[b], adv[b], aev[b]
        _wait_gathers(t, b)
        for k in range(CHUNK // 16):
            dtv[b][pl.ds(16 * k, 16)] = div[b][pl.ds(16 * k, 16)]
        if start_idx2:
            _start_idx(t + 2, b)

        @plsc.parallel_loop(0, CHUNK, unroll=8)
        def _edge(e):
            logit = rows_v[e, pl.ds(128, 16)] + adst_v[e, :] + ae_v[e, :]
            w = jnp.exp(jnp.maximum(logit, 0.2 * logit))
            rows_v[e, pl.ds(128, 16)] = w
            for j in range(8):
                ws = w[j // 2]
                rows_v[e, pl.ds(16 * j, 16)] = rows_v[e, pl.ds(16 * j, 16)] * ws

        pltpu.async_copy(rows_v, acc_sh.at[dtv[b]], sem_s[b], add=True)
        if start_next:
            if wait_prev_scat:
                _wait_scat(1 - b)
            _wait_idx(t + 1, 1 - b)
            _start_gathers(t + 1, 1 - b)

    def _pair(p, _):
        t = 2 * p + 2
        _chunk_body(t, 0, True, True, True)
        _chunk_body(t + 1, 1, True, True, True)
        return 0

    _start_idx(0, 0)
    _start_idx(1, 1)
    _wait_idx(0, 0)
    _start_gathers(0, 0)
    _chunk_body(0, 0, True, True, False)
    _chunk_body(1, 1, True, True, True)
    lax.fori_loop(0, (n_chunks - 4) // 2, _pair, 0)
    _chunk_body(n_chunks - 2, 0, False, True, True)
    _chunk_body(n_chunks - 1, 1, False, False, False)
    for b in range(2):
        _wait_scat(b)
    plsc.subcore_barrier()

    for k in range(nfull):
        rr = r0 + k * CHUNK
        pltpu.sync_copy(acc_sh.at[pl.ds(rr, CHUNK)], rows0)
        pltpu.sync_copy(rows0, out_hbm.at[c, pl.ds(rr, CHUNK)])
    if rem:
        rr = r0 + nfull * CHUNK
        pltpu.sync_copy(acc_sh.at[pl.ds(rr, rem)], rows0.at[pl.ds(0, rem)])
        pltpu.sync_copy(rows0.at[pl.ds(0, rem)], out_hbm.at[c, pl.ds(rr, rem)])


def _fin_body(parts_ref, bias_ref, g4_ref, wt0, bv0, wt1, bv1, wt2, bv2,
              lng, lnb, ho_ref, hs_ref, cs_ref):
    acc = parts_ref[0] + parts_ref[1] if parts_ref.shape[0] == 2 else parts_ref[0]
    den = jnp.dot(acc[:, 128:132], g4_ref[...], preferred_element_type=jnp.float32)
    sp = acc[:, :128] / (den + 1e-16) + bias_ref[...]

    def _lstm(inp, wt, bv):
        g = jnp.dot(inp, wt[...], preferred_element_type=jnp.float32) + bv[...]
        cc = jax.nn.sigmoid(g[:, 0:128]) * jnp.tanh(g[:, 256:384])
        hh = jax.nn.sigmoid(g[:, 384:512]) * jnp.tanh(cc)
        return hh, cc

    h1, c1 = _lstm(sp, wt0, bv0)
    h2, c2 = _lstm(h1, wt1, bv1)
    h3, c3 = _lstm(h2, wt2, bv2)
    mu = jnp.mean(h3, axis=1, keepdims=True)
    var = jnp.mean((h3 - mu) ** 2, axis=1, keepdims=True)
    ho_ref[...] = (h3 - mu) * lax.rsqrt(var + 1e-5) * lng[...] + lnb[...]
    hs_ref[0] = h1
    hs_ref[1] = h2
    hs_ref[2] = h3
    cs_ref[0] = c1
    cs_ref[1] = c2
    cs_ref[2] = c3


def kernel(x, edge_attr, W_lin, att_src, att_dst, W_edge, att_edge, bias_gat,
           W_ih0, W_hh0, b_ih0, b_hh0, W_ih1, W_hh1, b_ih1, b_hh1,
           W_ih2, W_hh2, b_ih2, b_hh2, ln_g, ln_b, edge_index):
    B, N, NF = x.shape
    E = edge_index.shape[1]
    ED = edge_attr.shape[2]
    HID = W_ih0.shape[1]
    f32 = jnp.float32

    BLK = 512
    N_PAD = -(-N // 2560) * 2560
    E_SELF = E + B * N
    E_PAD = -(-E_SELF // (NC * NS * CHUNK * 2)) * (NC * NS * CHUNK * 2)
    e_per_tile = E_PAD // (NC * NS)
    n_chunks = e_per_tile // CHUNK

    wlt = W_lin.T
    asrc_row = att_src.reshape(1, H * CO)
    adst_row = att_dst.reshape(1, H * CO)
    heads = jnp.arange(H * CO, dtype=jnp.int32) // CO
    gsum = (heads[:, None] == jnp.arange(16, dtype=jnp.int32)[None, :]).astype(f32)
    me = jnp.einsum('hcd,hc->dh', W_edge.reshape(H, CO, ED), att_edge[0])
    me16 = jnp.zeros((16, 16), f32).at[:ED, :H].set(me)
    me128 = jnp.kron(jnp.eye(8, dtype=f32), me16)
    g4 = (jnp.arange(4, dtype=jnp.int32)[:, None] == heads[None, :]).astype(f32)

    xp = jnp.zeros((N_PAD, NF), f32).at[:N].set(x.reshape(N, NF))
    loop = jnp.arange(N, dtype=edge_index.dtype)
    n_fill = E_PAD - E_SELF
    fill = (N + jnp.arange(n_fill, dtype=edge_index.dtype) % (N_PAD - N - 8))
    src = jnp.concatenate([edge_index[0], loop, fill])
    dst = jnp.concatenate([edge_index[1], loop, fill])
    eap = jnp.zeros((E_PAD, 16), f32).at[:E, :ED].set(edge_attr.reshape(E, ED))

    node_tab, adst_tab = pl.pallas_call(
        _prep_body,
        grid=(N_PAD // BLK,),
        in_specs=[
            pl.BlockSpec((BLK, NF), lambda i: (i, 0)),
            pl.BlockSpec((NF, H * CO), lambda i: (0, 0)),
            pl.BlockSpec((1, H * CO), lambda i: (0, 0)),
            pl.BlockSpec((1, H * CO), lambda i: (0, 0)),
            pl.BlockSpec((H * CO, 16), lambda i: (0, 0)),
        ],
        out_specs=[
            pl.BlockSpec((BLK, ROW), lambda i: (i, 0)),
            pl.BlockSpec((BLK, 16), lambda i: (i, 0)),
        ],
        out_shape=[
            jax.ShapeDtypeStruct((N_PAD, ROW), f32),
            jax.ShapeDtypeStruct((N_PAD, 16), f32),
        ],
    )(xp, wlt, asrc_row, adst_row, gsum)

    ea_wide = eap.reshape(E_PAD // 8, 128)
    ae_tab = pl.pallas_call(
        _ae_body,
        grid=(E_PAD // 8 // BLK,),
        in_specs=[
            pl.BlockSpec((BLK, 128), lambda i: (i, 0)),
            pl.BlockSpec((128, 128), lambda i: (0, 0)),
        ],
        out_specs=pl.BlockSpec((BLK, 128), lambda i: (i, 0)),
        out_shape=jax.ShapeDtypeStruct((E_PAD // 8, 128), f32),
    )(ea_wide, me128).reshape(E_PAD, 16)

    mesh = plsc.VectorSubcoreMesh(core_axis_name="c", subcore_axis_name="s",
                                  num_cores=NC)
    sc_call = pl.kernel(
        functools.partial(_sc_body, e_per_tile),
        out_type=jax.ShapeDtypeStruct((NC, N_PAD, ROW), f32),
        mesh=mesh,
        scratch_types=(
            [pltpu.VMEM_SHARED((N_PAD, ROW), f32)]
            + [pltpu.VMEM((CHUNK, ROW), f32)] * 2
            + [pltpu.VMEM((CHUNK, 16), f32)] * 4
            + [pltpu.VMEM((CHUNK,), jnp.int32)] * 6
            + [pltpu.SemaphoreType.DMA] * 12
        ),
        compiler_params=pltpu.CompilerParams(use_tc_tiling_on_sc=False),
    )
    parts = sc_call(node_tab, adst_tab, ae_tab, src, dst)

    wt0, wt1, wt2 = W_ih0.T, W_ih1.T, W_ih2.T
    bv0 = (b_ih0 + b_hh0).reshape(1, 4 * HID)
    bv1 = (b_ih1 + b_hh1).reshape(1, 4 * HID)
    bv2 = (b_ih2 + b_hh2).reshape(1, 4 * HID)
    n_blocks = -(-N // BLK)
    ho, hs, cs = pl.pallas_call(
        _fin_body,
        grid=(n_blocks,),
        in_specs=[
            pl.BlockSpec((NC, BLK, ROW), lambda i: (0, i, 0)),
            pl.BlockSpec((1, H * CO), lambda i: (0, 0)),
            pl.BlockSpec((4, 128), lambda i: (0, 0)),
            pl.BlockSpec((HID, 4 * HID), lambda i: (0, 0)),
            pl.BlockSpec((1, 4 * HID), lambda i: (0, 0)),
            pl.BlockSpec((HID, 4 * HID), lambda i: (0, 0)),
            pl.BlockSpec((1, 4 * HID), lambda i: (0, 0)),
            pl.BlockSpec((HID, 4 * HID), lambda i: (0, 0)),
            pl.BlockSpec((1, 4 * HID), lambda i: (0, 0)),
            pl.BlockSpec((1, HID), lambda i: (0, 0)),
            pl.BlockSpec((1, HID), lambda i: (0, 0)),
        ],
        out_specs=[
            pl.BlockSpec((BLK, HID), lambda i: (i, 0)),
            pl.BlockSpec((3, BLK, HID), lambda i: (0, i, 0)),
            pl.BlockSpec((3, BLK, HID), lambda i: (0, i, 0)),
        ],
        out_shape=[
            jax.ShapeDtypeStruct((N, HID), f32),
            jax.ShapeDtypeStruct((3, N, HID), f32),
            jax.ShapeDtypeStruct((3, N, HID), f32),
        ],
    )(parts, bias_gat.reshape(1, H * CO), g4, wt0, bv0, wt1, bv1, wt2, bv2,
      ln_g.reshape(1, HID), ln_b.reshape(1, HID))

    return (ho.reshape(B, N, HID), hs, cs)
